# baseline (speedup 1.0000x reference)
import functools

import jax
import jax.numpy as jnp
from jax import lax
from jax.experimental import pallas as pl
from jax.experimental.pallas import tpu as pltpu
from jax.experimental.pallas import tpu_sc as plsc

_N = 10000
_E = 320000
_F_IN = 128
_HID = 64
_C = 16

_NC = 2
_NS = 16
_NW = _NC * _NS
_CHUNK = 125
_ROWS = _E // _CHUNK
_ROWS_W = _ROWS // _NW
_NPAD = 10240
_NPS = _NPAD // _NS

_mesh = plsc.VectorSubcoreMesh(
    core_axis_name="c", subcore_axis_name="s", num_cores=_NC, num_subcores=_NS
)


@functools.partial(
    pl.kernel,
    out_type=[
        jax.ShapeDtypeStruct((_NPAD,), jnp.float32),
        jax.ShapeDtypeStruct((_NPAD,), jnp.float32),
    ],
    mesh=_mesh,
    compiler_params=pltpu.CompilerParams(use_tc_tiling_on_sc=False),
    scratch_types=[
        pltpu.VMEM((_ROWS_W, _CHUNK), jnp.int32),
        pltpu.VMEM((128,), jnp.float32),
        pltpu.VMEM_SHARED((_NPAD,), jnp.float32),
    ],
)
def _sc_degree(col_hbm, z1_hbm, cnt0_hbm, cnt1_hbm, colv, ones_v, acc):
    cid = lax.axis_index("c")
    sid = lax.axis_index("s")
    wid = sid * _NC + cid
    pltpu.sync_copy(z1_hbm.at[pl.ds(sid * _NPS, _NPS)], acc.at[pl.ds(sid * _NPS, _NPS)])
    pltpu.sync_copy(col_hbm.at[pl.ds(wid * _ROWS_W, _ROWS_W)], colv)
    for k in range(8):
        ones_v[pl.ds(k * 16, 16)] = jnp.ones((16,), jnp.float32)
    plsc.subcore_barrier()

    def body(j, carry):
        pltpu.sync_copy(ones_v.at[pl.ds(0, _CHUNK)], acc.at[colv.at[j]], add=True)
        return carry

    lax.fori_loop(0, _ROWS_W, body, 0)
    plsc.subcore_barrier()

    @pl.when(cid == 0)
    def _():
        pltpu.sync_copy(acc.at[pl.ds(sid * _NPS, _NPS)], cnt0_hbm.at[pl.ds(sid * _NPS, _NPS)])

    @pl.when(cid == 1)
    def _():
        pltpu.sync_copy(acc.at[pl.ds(sid * _NPS, _NPS)], cnt1_hbm.at[pl.ds(sid * _NPS, _NPS)])


def _make_sc_scatter(depth):
    @functools.partial(
        pl.kernel,
        out_type=[
            jax.ShapeDtypeStruct((_NPAD, depth), jnp.float32),
            jax.ShapeDtypeStruct((_NPAD, depth), jnp.float32),
        ],
        mesh=_mesh,
        compiler_params=pltpu.CompilerParams(use_tc_tiling_on_sc=False),
        scratch_types=[
            pltpu.VMEM((_ROWS_W, _CHUNK), jnp.int32),
            pltpu.VMEM((_ROWS_W, _CHUNK), jnp.int32),
            pltpu.VMEM((_CHUNK, depth), jnp.float32),
            pltpu.VMEM((_CHUNK, depth), jnp.float32),
            pltpu.VMEM((_CHUNK, depth), jnp.float32),
            pltpu.VMEM((_CHUNK, depth), jnp.float32),
            pltpu.VMEM_SHARED((_NPAD, depth), jnp.float32),
            pltpu.SemaphoreType.DMA,
            pltpu.SemaphoreType.DMA,
            pltpu.SemaphoreType.DMA,
            pltpu.SemaphoreType.DMA,
        ],
    )
    def _sc_scatter(row_hbm, col_hbm, y_hbm, zd_hbm, p0_hbm, p1_hbm,
                    rowv, colv, buf_a, buf_b, buf_c, buf_d, acc,
                    sem_a, sem_b, sem_c, sem_d):
        cid = lax.axis_index("c")
        sid = lax.axis_index("s")
        wid = sid * _NC + cid
        bufs = (buf_a, buf_b, buf_c, buf_d)
        sems = (sem_a, sem_b, sem_c, sem_d)
        nbuf = 4
        pltpu.sync_copy(zd_hbm.at[pl.ds(sid * _NPS, _NPS)],
                        acc.at[pl.ds(sid * _NPS, _NPS)])
        pltpu.sync_copy(row_hbm.at[pl.ds(wid * _ROWS_W, _ROWS_W)], rowv)
        pltpu.sync_copy(col_hbm.at[pl.ds(wid * _ROWS_W, _ROWS_W)], colv)
        plsc.subcore_barrier()

        for b in range(nbuf):
            pltpu.async_copy(y_hbm.at[rowv.at[b]], bufs[b], sems[b])

        def body(i, carry):
            for b in range(nbuf):
                j = nbuf * i + b
                pltpu.make_async_copy(y_hbm.at[rowv.at[j]], bufs[b], sems[b]).wait()
                pltpu.sync_copy(bufs[b], acc.at[colv.at[j]], add=True)

                @pl.when(j + nbuf < _ROWS_W)
                def _():
                    pltpu.async_copy(y_hbm.at[rowv.at[j + nbuf]], bufs[b], sems[b])

            return carry

        lax.fori_loop(0, _ROWS_W // nbuf, body, 0)
        plsc.subcore_barrier()

        @pl.when(cid == 0)
        def _():
            pltpu.sync_copy(acc.at[pl.ds(sid * _NPS, _NPS)],
                            p0_hbm.at[pl.ds(sid * _NPS, _NPS)])

        @pl.when(cid == 1)
        def _():
            pltpu.sync_copy(acc.at[pl.ds(sid * _NPS, _NPS)],
                            p1_hbm.at[pl.ds(sid * _NPS, _NPS)])

    return _sc_scatter


_sc_scatter_hid = _make_sc_scatter(_HID)
_sc_scatter_out = _make_sc_scatter(_C)


_R = 1000
_G = _N // _R


def _tc1_body(x_ref, w1_ref, c0_ref, c1_ref, w2_ref, y_ref, dinv_ref, o_ref):
    deg = c0_ref[...] + c1_ref[...] + 1.0
    dinv = lax.rsqrt(deg)
    xw = jnp.dot(x_ref[...], w1_ref[...], preferred_element_type=jnp.float32)
    y_ref[...] = xw * dinv
    dinv_ref[...] = dinv

    @pl.when(pl.program_id(0) == 0)
    def _():
        w1 = w1_ref[...]
        w2 = w2_ref[...]
        g1 = lax.dot_general(w1, w1, (((1,), (1,)), ((), ())),
                             preferred_element_type=jnp.float32)
        g2 = lax.dot_general(w2, w2, (((1,), (1,)), ((), ())),
                             preferred_element_type=jnp.float32)
        i1 = (lax.broadcasted_iota(jnp.int32, (_F_IN, _F_IN), 0)
              == lax.broadcasted_iota(jnp.int32, (_F_IN, _F_IN), 1)).astype(jnp.float32)
        i2 = (lax.broadcasted_iota(jnp.int32, (_HID, _HID), 0)
              == lax.broadcasted_iota(jnp.int32, (_HID, _HID), 1)).astype(jnp.float32)
        s1 = jnp.sum((g1 - i1) ** 2)
        s2 = jnp.sum((g2 - i2) ** 2)
        o_ref[...] = jnp.reshape(jnp.sqrt(s1) + jnp.sqrt(s2), (1, 1))


_tc1 = pl.pallas_call(
    _tc1_body,
    grid=(_G,),
    in_specs=[
        pl.BlockSpec((_R, _F_IN), lambda i: (i, 0)),
        pl.BlockSpec((_F_IN, _HID), lambda i: (0, 0)),
        pl.BlockSpec((_R, 1), lambda i: (i, 0)),
        pl.BlockSpec((_R, 1), lambda i: (i, 0)),
        pl.BlockSpec((_HID, _C), lambda i: (0, 0)),
    ],
    out_specs=[
        pl.BlockSpec((_R, _HID), lambda i: (i, 0)),
        pl.BlockSpec((_R, 1), lambda i: (i, 0)),
        pl.BlockSpec((1, 1), lambda i: (0, 0)),
    ],
    out_shape=[
        jax.ShapeDtypeStruct((_N, _HID), jnp.float32),
        jax.ShapeDtypeStruct((_N, 1), jnp.float32),
        jax.ShapeDtypeStruct((1, 1), jnp.float32),
    ],
)


def _tc2_body(p0_ref, p1_ref, y1_ref, dinv_ref, b1_ref, w2_ref, z_ref):
    dinv = dinv_ref[...]
    out1 = (p0_ref[...] + p1_ref[...] + y1_ref[...]) * dinv + b1_ref[...]
    h = jnp.maximum(out1, 0.0)
    z_ref[...] = jnp.dot(h, w2_ref[...], preferred_element_type=jnp.float32) * dinv


_tc2 = pl.pallas_call(
    _tc2_body,
    grid=(_G,),
    in_specs=[
        pl.BlockSpec((_R, _HID), lambda i: (i, 0)),
        pl.BlockSpec((_R, _HID), lambda i: (i, 0)),
        pl.BlockSpec((_R, _HID), lambda i: (i, 0)),
        pl.BlockSpec((_R, 1), lambda i: (i, 0)),
        pl.BlockSpec((1, _HID), lambda i: (0, 0)),
        pl.BlockSpec((_HID, _C), lambda i: (0, 0)),
    ],
    out_specs=[pl.BlockSpec((_R, _C), lambda i: (i, 0))],
    out_shape=[jax.ShapeDtypeStruct((_N, _C), jnp.float32)],
)


def _tc3_body(q0_ref, q1_ref, z2_ref, dinv_ref, b2_ref, logp_ref, xout_ref):
    xo = (q0_ref[...] + q1_ref[...] + z2_ref[...]) * dinv_ref[...] + b2_ref[...]
    m = jnp.max(xo, axis=1, keepdims=True)
    t = xo - m
    lse = jnp.log(jnp.sum(jnp.exp(t), axis=1, keepdims=True))
    logp_ref[...] = t - lse
    xout_ref[...] = xo


_tc3 = pl.pallas_call(
    _tc3_body,
    grid=(_G,),
    in_specs=[
        pl.BlockSpec((_R, _C), lambda i: (i, 0)),
        pl.BlockSpec((_R, _C), lambda i: (i, 0)),
        pl.BlockSpec((_R, _C), lambda i: (i, 0)),
        pl.BlockSpec((_R, 1), lambda i: (i, 0)),
        pl.BlockSpec((1, _C), lambda i: (0, 0)),
    ],
    out_specs=[
        pl.BlockSpec((_R, _C), lambda i: (i, 0)),
        pl.BlockSpec((_R, _C), lambda i: (i, 0)),
    ],
    out_shape=[
        jax.ShapeDtypeStruct((_N, _C), jnp.float32),
        jax.ShapeDtypeStruct((_N, _C), jnp.float32),
    ],
)


def kernel(x, edge_index, W1, b1, W2, b2):
    row = edge_index[0].reshape(_ROWS, _CHUNK)
    col = edge_index[1].reshape(_ROWS, _CHUNK)
    z1 = jnp.zeros((_NPAD,), jnp.float32)
    z64 = jnp.zeros((_NPAD, _HID), jnp.float32)
    z16 = jnp.zeros((_NPAD, _C), jnp.float32)

    cnt0, cnt1 = _sc_degree(col, z1)
    y1, dinv, orto = _tc1(x, W1, cnt0.reshape(_NPAD, 1), cnt1.reshape(_NPAD, 1), W2)
    p0, p1 = _sc_scatter_hid(row, col, y1, z64)
    (z2,) = _tc2(p0, p1, y1, dinv, b1.reshape(1, _HID), W2)
    q0, q1 = _sc_scatter_out(row, col, z2, z16)
    logp, xout = _tc3(q0, q1, z2, dinv, b2.reshape(1, _C))
    return (logp, xout, orto.reshape(()))


# speedup vs baseline: 1.0234x; 1.0234x over previous
import functools

import jax
import jax.numpy as jnp
from jax import lax
from jax.experimental import pallas as pl
from jax.experimental.pallas import tpu as pltpu
from jax.experimental.pallas import tpu_sc as plsc

_N = 10000
_E = 320000
_F_IN = 128
_HID = 64
_C = 16

_NC = 2
_NS = 16
_NW = _NC * _NS
_CHUNK = 125
_ROWS = _E // _CHUNK
_ROWS_W = _ROWS // _NW
_NPAD = 10240
_NPS = _NPAD // _NS

_mesh = plsc.VectorSubcoreMesh(
    core_axis_name="c", subcore_axis_name="s", num_cores=_NC, num_subcores=_NS
)


@functools.partial(
    pl.kernel,
    out_type=[
        jax.ShapeDtypeStruct((_NPAD,), jnp.float32),
        jax.ShapeDtypeStruct((_NPAD,), jnp.float32),
    ],
    mesh=_mesh,
    compiler_params=pltpu.CompilerParams(use_tc_tiling_on_sc=False),
    scratch_types=[
        pltpu.VMEM((_ROWS_W, _CHUNK), jnp.int32),
        pltpu.VMEM((128,), jnp.float32),
        pltpu.VMEM((_NPS,), jnp.float32),
        pltpu.VMEM_SHARED((_NPAD,), jnp.float32),
    ],
)
def _sc_degree(edge_hbm, cnt0_hbm, cnt1_hbm, colv, ones_v, zbuf, acc):
    cid = lax.axis_index("c")
    sid = lax.axis_index("s")
    wid = sid * _NC + cid
    for k in range(8):
        ones_v[pl.ds(k * 16, 16)] = jnp.ones((16,), jnp.float32)

    def zfill(k, carry):
        zbuf[pl.ds(k * 16, 16)] = jnp.zeros((16,), jnp.float32)
        return carry

    lax.fori_loop(0, _NPS // 16, zfill, 0)
    pltpu.sync_copy(zbuf, acc.at[pl.ds(sid * _NPS, _NPS)])
    pltpu.sync_copy(edge_hbm.at[1, pl.ds(wid * _ROWS_W, _ROWS_W)], colv)
    plsc.subcore_barrier()

    def body(j, carry):
        pltpu.sync_copy(ones_v.at[pl.ds(0, _CHUNK)], acc.at[colv.at[j]], add=True)
        return carry

    lax.fori_loop(0, _ROWS_W, body, 0)
    plsc.subcore_barrier()

    @pl.when(cid == 0)
    def _():
        pltpu.sync_copy(acc.at[pl.ds(sid * _NPS, _NPS)], cnt0_hbm.at[pl.ds(sid * _NPS, _NPS)])

    @pl.when(cid == 1)
    def _():
        pltpu.sync_copy(acc.at[pl.ds(sid * _NPS, _NPS)], cnt1_hbm.at[pl.ds(sid * _NPS, _NPS)])


def _make_sc_scatter(depth):
    @functools.partial(
        pl.kernel,
        out_type=[
            jax.ShapeDtypeStruct((_NPAD, depth), jnp.float32),
            jax.ShapeDtypeStruct((_NPAD, depth), jnp.float32),
        ],
        mesh=_mesh,
        compiler_params=pltpu.CompilerParams(use_tc_tiling_on_sc=False),
        scratch_types=[
            pltpu.VMEM((_ROWS_W, _CHUNK), jnp.int32),
            pltpu.VMEM((_ROWS_W, _CHUNK), jnp.int32),
            pltpu.VMEM((_NPS, depth), jnp.float32),
            pltpu.VMEM((_CHUNK, depth), jnp.float32),
            pltpu.VMEM((_CHUNK, depth), jnp.float32),
            pltpu.VMEM((_CHUNK, depth), jnp.float32),
            pltpu.VMEM((_CHUNK, depth), jnp.float32),
            pltpu.VMEM_SHARED((_NPAD, depth), jnp.float32),
            pltpu.SemaphoreType.DMA,
            pltpu.SemaphoreType.DMA,
            pltpu.SemaphoreType.DMA,
            pltpu.SemaphoreType.DMA,
        ],
    )
    def _sc_scatter(edge_hbm, y_hbm, zd_hbm, p0_hbm, p1_hbm,
                    rowv, colv, zbuf, buf_a, buf_b, buf_c, buf_d, acc,
                    sem_a, sem_b, sem_c, sem_d):
        cid = lax.axis_index("c")
        sid = lax.axis_index("s")
        wid = sid * _NC + cid
        bufs = (buf_a, buf_b, buf_c, buf_d)
        sems = (sem_a, sem_b, sem_c, sem_d)
        nbuf = 4

        pltpu.sync_copy(zd_hbm.at[pl.ds(sid * _NPS, _NPS)],
                        acc.at[pl.ds(sid * _NPS, _NPS)])
        pltpu.sync_copy(edge_hbm.at[0, pl.ds(wid * _ROWS_W, _ROWS_W)], rowv)
        pltpu.sync_copy(edge_hbm.at[1, pl.ds(wid * _ROWS_W, _ROWS_W)], colv)
        plsc.subcore_barrier()

        for b in range(nbuf):
            pltpu.async_copy(y_hbm.at[rowv.at[b]], bufs[b], sems[b])

        def body(i, carry):
            for b in range(nbuf):
                j = nbuf * i + b
                pltpu.make_async_copy(y_hbm.at[rowv.at[j]], bufs[b], sems[b]).wait()
                pltpu.sync_copy(bufs[b], acc.at[colv.at[j]], add=True)

                @pl.when(j + nbuf < _ROWS_W)
                def _():
                    pltpu.async_copy(y_hbm.at[rowv.at[j + nbuf]], bufs[b], sems[b])

            return carry

        lax.fori_loop(0, _ROWS_W // nbuf, body, 0)
        plsc.subcore_barrier()

        @pl.when(cid == 0)
        def _():
            pltpu.sync_copy(acc.at[pl.ds(sid * _NPS, _NPS)],
                            p0_hbm.at[pl.ds(sid * _NPS, _NPS)])

        @pl.when(cid == 1)
        def _():
            pltpu.sync_copy(acc.at[pl.ds(sid * _NPS, _NPS)],
                            p1_hbm.at[pl.ds(sid * _NPS, _NPS)])

    return _sc_scatter


_sc_scatter_hid = _make_sc_scatter(_HID)
_sc_scatter_out = _make_sc_scatter(_C)


_R = 1000
_G = _N // _R


def _tc1a_body(x_ref, w1_ref, w2_ref, c0_ref, c1_ref, y_ref, dinv_ref, o_ref):
    xw = jnp.dot(x_ref[...], w1_ref[...], preferred_element_type=jnp.float32)
    deg = c0_ref[...] + c1_ref[...] + 1.0
    dinv = lax.rsqrt(deg)
    y_ref[...] = xw * dinv
    dinv_ref[...] = dinv

    @pl.when(pl.program_id(0) == 0)
    def _():
        w1 = w1_ref[...]
        w2 = w2_ref[...]
        g1 = lax.dot_general(w1, w1, (((1,), (1,)), ((), ())),
                             preferred_element_type=jnp.float32)
        g2 = lax.dot_general(w2, w2, (((1,), (1,)), ((), ())),
                             preferred_element_type=jnp.float32)
        i1 = (lax.broadcasted_iota(jnp.int32, (_F_IN, _F_IN), 0)
              == lax.broadcasted_iota(jnp.int32, (_F_IN, _F_IN), 1)).astype(jnp.float32)
        i2 = (lax.broadcasted_iota(jnp.int32, (_HID, _HID), 0)
              == lax.broadcasted_iota(jnp.int32, (_HID, _HID), 1)).astype(jnp.float32)
        s1 = jnp.sum((g1 - i1) ** 2)
        s2 = jnp.sum((g2 - i2) ** 2)
        o_ref[...] = jnp.reshape(jnp.sqrt(s1) + jnp.sqrt(s2), (1, 1))


_tc1a = pl.pallas_call(
    _tc1a_body,
    grid=(_G,),
    in_specs=[
        pl.BlockSpec((_R, _F_IN), lambda i: (i, 0)),
        pl.BlockSpec((_F_IN, _HID), lambda i: (0, 0)),
        pl.BlockSpec((_HID, _C), lambda i: (0, 0)),
        pl.BlockSpec((_R, 1), lambda i: (i, 0)),
        pl.BlockSpec((_R, 1), lambda i: (i, 0)),
    ],
    out_specs=[
        pl.BlockSpec((_R, _HID), lambda i: (i, 0)),
        pl.BlockSpec((_R, 1), lambda i: (i, 0)),
        pl.BlockSpec((1, 1), lambda i: (0, 0)),
    ],
    out_shape=[
        jax.ShapeDtypeStruct((_N, _HID), jnp.float32),
        jax.ShapeDtypeStruct((_N, 1), jnp.float32),
        jax.ShapeDtypeStruct((1, 1), jnp.float32),
    ],
)


def _tc2_body(p0_ref, p1_ref, y1_ref, dinv_ref, b1_ref, w2_ref, z_ref):
    dinv = dinv_ref[...]
    out1 = (p0_ref[...] + p1_ref[...] + y1_ref[...]) * dinv + b1_ref[...]
    h = jnp.maximum(out1, 0.0)
    z_ref[...] = jnp.dot(h, w2_ref[...], preferred_element_type=jnp.float32) * dinv


_tc2 = pl.pallas_call(
    _tc2_body,
    grid=(_G,),
    in_specs=[
        pl.BlockSpec((_R, _HID), lambda i: (i, 0)),
        pl.BlockSpec((_R, _HID), lambda i: (i, 0)),
        pl.BlockSpec((_R, _HID), lambda i: (i, 0)),
        pl.BlockSpec((_R, 1), lambda i: (i, 0)),
        pl.BlockSpec((1, _HID), lambda i: (0, 0)),
        pl.BlockSpec((_HID, _C), lambda i: (0, 0)),
    ],
    out_specs=[pl.BlockSpec((_R, _C), lambda i: (i, 0))],
    out_shape=[jax.ShapeDtypeStruct((_N, _C), jnp.float32)],
)


def _tc3_body(q0_ref, q1_ref, z2_ref, dinv_ref, b2_ref, logp_ref, xout_ref):
    xo = (q0_ref[...] + q1_ref[...] + z2_ref[...]) * dinv_ref[...] + b2_ref[...]
    m = jnp.max(xo, axis=1, keepdims=True)
    t = xo - m
    lse = jnp.log(jnp.sum(jnp.exp(t), axis=1, keepdims=True))
    logp_ref[...] = t - lse
    xout_ref[...] = xo


_tc3 = pl.pallas_call(
    _tc3_body,
    grid=(_G,),
    in_specs=[
        pl.BlockSpec((_R, _C), lambda i: (i, 0)),
        pl.BlockSpec((_R, _C), lambda i: (i, 0)),
        pl.BlockSpec((_R, _C), lambda i: (i, 0)),
        pl.BlockSpec((_R, 1), lambda i: (i, 0)),
        pl.BlockSpec((1, _C), lambda i: (0, 0)),
    ],
    out_specs=[
        pl.BlockSpec((_R, _C), lambda i: (i, 0)),
        pl.BlockSpec((_R, _C), lambda i: (i, 0)),
    ],
    out_shape=[
        jax.ShapeDtypeStruct((_N, _C), jnp.float32),
        jax.ShapeDtypeStruct((_N, _C), jnp.float32),
    ],
)


def kernel(x, edge_index, W1, b1, W2, b2):
    edge3 = edge_index.reshape(2, _ROWS, _CHUNK)

    cnt0, cnt1 = _sc_degree(edge3)
    y1, dinv, orto = _tc1a(x, W1, W2, cnt0.reshape(_NPAD, 1)[: _N],
                           cnt1.reshape(_NPAD, 1)[: _N])
    z64 = jnp.zeros((_NPAD, _HID), jnp.float32)
    z16 = jnp.zeros((_NPAD, _C), jnp.float32)
    p0, p1 = _sc_scatter_hid(edge3, y1, z64)
    (z2,) = _tc2(p0, p1, y1, dinv, b1.reshape(1, _HID), W2)
    q0, q1 = _sc_scatter_out(edge3, z2, z16)
    logp, xout = _tc3(q0, q1, z2, dinv, b2.reshape(1, _C))
    return (logp, xout, orto.reshape(()))


# speedup vs baseline: 1.0443x; 1.0205x over previous
import functools

import jax
import jax.numpy as jnp
from jax import lax
from jax.experimental import pallas as pl
from jax.experimental.pallas import tpu as pltpu
from jax.experimental.pallas import tpu_sc as plsc

_N = 10000
_E = 320000
_F_IN = 128
_HID = 64
_C = 16

_NC = 2
_NS = 16
_NW = _NC * _NS
_CHUNK = 125
_ROWS = _E // _CHUNK
_ROWS_W = _ROWS // _NW
_NPAD = 10240
_NPS = _NPAD // _NS

_mesh = plsc.VectorSubcoreMesh(
    core_axis_name="c", subcore_axis_name="s", num_cores=_NC, num_subcores=_NS
)


@functools.partial(
    pl.kernel,
    out_type=[
        jax.ShapeDtypeStruct((_NPAD,), jnp.float32),
        jax.ShapeDtypeStruct((_NPAD,), jnp.float32),
    ],
    mesh=_mesh,
    compiler_params=pltpu.CompilerParams(use_tc_tiling_on_sc=False),
    scratch_types=[
        pltpu.VMEM((_ROWS_W, _CHUNK), jnp.int32),
        pltpu.VMEM((128,), jnp.float32),
        pltpu.VMEM((_NPS,), jnp.float32),
        pltpu.VMEM_SHARED((_NPAD,), jnp.float32),
    ],
)
def _sc_degree(edge_hbm, cnt0_hbm, cnt1_hbm, colv, ones_v, zbuf, acc):
    cid = lax.axis_index("c")
    sid = lax.axis_index("s")
    wid = sid * _NC + cid
    for k in range(8):
        ones_v[pl.ds(k * 16, 16)] = jnp.ones((16,), jnp.float32)

    def zfill(k, carry):
        zbuf[pl.ds(k * 16, 16)] = jnp.zeros((16,), jnp.float32)
        return carry

    lax.fori_loop(0, _NPS // 16, zfill, 0)
    pltpu.sync_copy(zbuf, acc.at[pl.ds(sid * _NPS, _NPS)])
    pltpu.sync_copy(edge_hbm.at[1, pl.ds(wid * _ROWS_W, _ROWS_W)], colv)
    plsc.subcore_barrier()

    def body(j, carry):
        pltpu.sync_copy(ones_v.at[pl.ds(0, _CHUNK)], acc.at[colv.at[j]], add=True)
        return carry

    lax.fori_loop(0, _ROWS_W, body, 0)
    plsc.subcore_barrier()

    @pl.when(cid == 0)
    def _():
        pltpu.sync_copy(acc.at[pl.ds(sid * _NPS, _NPS)], cnt0_hbm.at[pl.ds(sid * _NPS, _NPS)])

    @pl.when(cid == 1)
    def _():
        pltpu.sync_copy(acc.at[pl.ds(sid * _NPS, _NPS)], cnt1_hbm.at[pl.ds(sid * _NPS, _NPS)])


def _make_sc_scatter(depth, chunk, nbuf):
    rows = _E // chunk
    rows_w = rows // _NW

    @functools.partial(
        pl.kernel,
        out_type=[
            jax.ShapeDtypeStruct((_NPAD, depth), jnp.float32),
            jax.ShapeDtypeStruct((_NPAD, depth), jnp.float32),
        ],
        mesh=_mesh,
        compiler_params=pltpu.CompilerParams(use_tc_tiling_on_sc=False),
        scratch_types=(
            [
                pltpu.VMEM((rows_w, chunk), jnp.int32),
                pltpu.VMEM((rows_w, chunk), jnp.int32),
            ]
            + [pltpu.VMEM((chunk, depth), jnp.float32)] * nbuf
            + [pltpu.VMEM_SHARED((_NPAD, depth), jnp.float32)]
            + [pltpu.SemaphoreType.DMA] * nbuf
        ),
    )
    def _sc_scatter(edge_hbm, y_hbm, zd_hbm, p0_hbm, p1_hbm,
                    rowv, colv, *rest):
        bufs = rest[:nbuf]
        acc = rest[nbuf]
        sems = rest[nbuf + 1:]
        cid = lax.axis_index("c")
        sid = lax.axis_index("s")
        wid = sid * _NC + cid

        pltpu.sync_copy(zd_hbm.at[pl.ds(sid * _NPS, _NPS)],
                        acc.at[pl.ds(sid * _NPS, _NPS)])
        pltpu.sync_copy(edge_hbm.at[0, pl.ds(wid * rows_w, rows_w)], rowv)
        pltpu.sync_copy(edge_hbm.at[1, pl.ds(wid * rows_w, rows_w)], colv)
        plsc.subcore_barrier()

        for b in range(nbuf):
            pltpu.async_copy(y_hbm.at[rowv.at[b]], bufs[b], sems[b])

        def body(i, carry):
            for b in range(nbuf):
                j = nbuf * i + b
                pltpu.make_async_copy(y_hbm.at[rowv.at[j]], bufs[b], sems[b]).wait()
                pltpu.sync_copy(bufs[b], acc.at[colv.at[j]], add=True)

                @pl.when(j + nbuf < rows_w)
                def _():
                    pltpu.async_copy(y_hbm.at[rowv.at[j + nbuf]], bufs[b], sems[b])

            return carry

        lax.fori_loop(0, rows_w // nbuf, body, 0)
        plsc.subcore_barrier()

        @pl.when(cid == 0)
        def _():
            pltpu.sync_copy(acc.at[pl.ds(sid * _NPS, _NPS)],
                            p0_hbm.at[pl.ds(sid * _NPS, _NPS)])

        @pl.when(cid == 1)
        def _():
            pltpu.sync_copy(acc.at[pl.ds(sid * _NPS, _NPS)],
                            p1_hbm.at[pl.ds(sid * _NPS, _NPS)])

    return _sc_scatter


_CHUNK1 = 250
_CHUNK2 = 500
_sc_scatter_hid = _make_sc_scatter(_HID, _CHUNK1, 4)
_sc_scatter_out = _make_sc_scatter(_C, _CHUNK2, 4)


_R = 1000
_G = _N // _R


def _tc1a_body(x_ref, w1_ref, w2_ref, c0_ref, c1_ref, y_ref, dinv_ref, o_ref):
    xw = jnp.dot(x_ref[...], w1_ref[...], preferred_element_type=jnp.float32)
    deg = c0_ref[...] + c1_ref[...] + 1.0
    dinv = lax.rsqrt(deg)
    y_ref[...] = xw * dinv
    dinv_ref[...] = dinv

    @pl.when(pl.program_id(0) == 0)
    def _():
        w1 = w1_ref[...]
        w2 = w2_ref[...]
        g1 = lax.dot_general(w1, w1, (((1,), (1,)), ((), ())),
                             preferred_element_type=jnp.float32)
        g2 = lax.dot_general(w2, w2, (((1,), (1,)), ((), ())),
                             preferred_element_type=jnp.float32)
        i1 = (lax.broadcasted_iota(jnp.int32, (_F_IN, _F_IN), 0)
              == lax.broadcasted_iota(jnp.int32, (_F_IN, _F_IN), 1)).astype(jnp.float32)
        i2 = (lax.broadcasted_iota(jnp.int32, (_HID, _HID), 0)
              == lax.broadcasted_iota(jnp.int32, (_HID, _HID), 1)).astype(jnp.float32)
        s1 = jnp.sum((g1 - i1) ** 2)
        s2 = jnp.sum((g2 - i2) ** 2)
        o_ref[...] = jnp.reshape(jnp.sqrt(s1) + jnp.sqrt(s2), (1, 1))


_tc1a = pl.pallas_call(
    _tc1a_body,
    grid=(_G,),
    in_specs=[
        pl.BlockSpec((_R, _F_IN), lambda i: (i, 0)),
        pl.BlockSpec((_F_IN, _HID), lambda i: (0, 0)),
        pl.BlockSpec((_HID, _C), lambda i: (0, 0)),
        pl.BlockSpec((_R, 1), lambda i: (i, 0)),
        pl.BlockSpec((_R, 1), lambda i: (i, 0)),
    ],
    out_specs=[
        pl.BlockSpec((_R, _HID), lambda i: (i, 0)),
        pl.BlockSpec((_R, 1), lambda i: (i, 0)),
        pl.BlockSpec((1, 1), lambda i: (0, 0)),
    ],
    out_shape=[
        jax.ShapeDtypeStruct((_N, _HID), jnp.float32),
        jax.ShapeDtypeStruct((_N, 1), jnp.float32),
        jax.ShapeDtypeStruct((1, 1), jnp.float32),
    ],
)


def _tc2_body(p0_ref, p1_ref, y1_ref, dinv_ref, b1_ref, w2_ref, z_ref):
    dinv = dinv_ref[...]
    out1 = (p0_ref[...] + p1_ref[...] + y1_ref[...]) * dinv + b1_ref[...]
    h = jnp.maximum(out1, 0.0)
    z_ref[...] = jnp.dot(h, w2_ref[...], preferred_element_type=jnp.float32) * dinv


_tc2 = pl.pallas_call(
    _tc2_body,
    grid=(_G,),
    in_specs=[
        pl.BlockSpec((_R, _HID), lambda i: (i, 0)),
        pl.BlockSpec((_R, _HID), lambda i: (i, 0)),
        pl.BlockSpec((_R, _HID), lambda i: (i, 0)),
        pl.BlockSpec((_R, 1), lambda i: (i, 0)),
        pl.BlockSpec((1, _HID), lambda i: (0, 0)),
        pl.BlockSpec((_HID, _C), lambda i: (0, 0)),
    ],
    out_specs=[pl.BlockSpec((_R, _C), lambda i: (i, 0))],
    out_shape=[jax.ShapeDtypeStruct((_N, _C), jnp.float32)],
)


def _tc3_body(q0_ref, q1_ref, z2_ref, dinv_ref, b2_ref, logp_ref, xout_ref):
    xo = (q0_ref[...] + q1_ref[...] + z2_ref[...]) * dinv_ref[...] + b2_ref[...]
    m = jnp.max(xo, axis=1, keepdims=True)
    t = xo - m
    lse = jnp.log(jnp.sum(jnp.exp(t), axis=1, keepdims=True))
    logp_ref[...] = t - lse
    xout_ref[...] = xo


_tc3 = pl.pallas_call(
    _tc3_body,
    grid=(_G,),
    in_specs=[
        pl.BlockSpec((_R, _C), lambda i: (i, 0)),
        pl.BlockSpec((_R, _C), lambda i: (i, 0)),
        pl.BlockSpec((_R, _C), lambda i: (i, 0)),
        pl.BlockSpec((_R, 1), lambda i: (i, 0)),
        pl.BlockSpec((1, _C), lambda i: (0, 0)),
    ],
    out_specs=[
        pl.BlockSpec((_R, _C), lambda i: (i, 0)),
        pl.BlockSpec((_R, _C), lambda i: (i, 0)),
    ],
    out_shape=[
        jax.ShapeDtypeStruct((_N, _C), jnp.float32),
        jax.ShapeDtypeStruct((_N, _C), jnp.float32),
    ],
)


def kernel(x, edge_index, W1, b1, W2, b2):
    edge3 = edge_index.reshape(2, _ROWS, _CHUNK)

    cnt0, cnt1 = _sc_degree(edge3)
    y1, dinv, orto = _tc1a(x, W1, W2, cnt0.reshape(_NPAD, 1)[: _N],
                           cnt1.reshape(_NPAD, 1)[: _N])
    z64 = jnp.zeros((_NPAD, _HID), jnp.float32)
    z16 = jnp.zeros((_NPAD, _C), jnp.float32)
    p0, p1 = _sc_scatter_hid(edge_index.reshape(2, _E // _CHUNK1, _CHUNK1), y1, z64)
    (z2,) = _tc2(p0, p1, y1, dinv, b1.reshape(1, _HID), W2)
    q0, q1 = _sc_scatter_out(edge_index.reshape(2, _E // _CHUNK2, _CHUNK2), z2, z16)
    logp, xout = _tc3(q0, q1, z2, dinv, b2.reshape(1, _C))
    return (logp, xout, orto.reshape(()))


# speedup vs baseline: 1.0828x; 1.0368x over previous
import functools

import jax
import jax.numpy as jnp
from jax import lax
from jax.experimental import pallas as pl
from jax.experimental.pallas import tpu as pltpu
from jax.experimental.pallas import tpu_sc as plsc

_N = 10000
_E = 320000
_F_IN = 128
_HID = 64
_C = 16

_NC = 2
_NS = 16
_NW = _NC * _NS
_CHUNK = 125
_ROWS = _E // _CHUNK
_ROWS_W = _ROWS // _NW
_NPAD = 10240
_NPS = _NPAD // _NS

_ECHUNK = 500
_EROWS = _E // _ECHUNK
_EROWS_W = _EROWS // _NW

_mesh = plsc.VectorSubcoreMesh(
    core_axis_name="c", subcore_axis_name="s", num_cores=_NC, num_subcores=_NS
)


@functools.partial(
    pl.kernel,
    out_type=[
        jax.ShapeDtypeStruct((_NPAD,), jnp.float32),
        jax.ShapeDtypeStruct((_NPAD,), jnp.float32),
    ],
    mesh=_mesh,
    compiler_params=pltpu.CompilerParams(use_tc_tiling_on_sc=False),
    scratch_types=[
        pltpu.VMEM((_EROWS_W, _ECHUNK), jnp.int32),
        pltpu.VMEM((512,), jnp.float32),
        pltpu.VMEM((_NPS,), jnp.float32),
        pltpu.VMEM_SHARED((_NPAD,), jnp.float32),
    ],
)
def _sc_degree(edge_hbm, cnt0_hbm, cnt1_hbm, colv, ones_v, zbuf, acc):
    cid = lax.axis_index("c")
    sid = lax.axis_index("s")
    wid = sid * _NC + cid
    for k in range(32):
        ones_v[pl.ds(k * 16, 16)] = jnp.ones((16,), jnp.float32)

    def zfill(k, carry):
        zbuf[pl.ds(k * 16, 16)] = jnp.zeros((16,), jnp.float32)
        return carry

    lax.fori_loop(0, _NPS // 16, zfill, 0)
    pltpu.sync_copy(zbuf, acc.at[pl.ds(sid * _NPS, _NPS)])
    pltpu.sync_copy(edge_hbm.at[1, pl.ds(wid * _EROWS_W, _EROWS_W)], colv)
    plsc.subcore_barrier()

    def body(j, carry):
        pltpu.sync_copy(ones_v.at[pl.ds(0, _ECHUNK)], acc.at[colv.at[j]], add=True)
        return carry

    lax.fori_loop(0, _EROWS_W, body, 0)
    plsc.subcore_barrier()

    @pl.when(cid == 0)
    def _():
        pltpu.sync_copy(acc.at[pl.ds(sid * _NPS, _NPS)], cnt0_hbm.at[pl.ds(sid * _NPS, _NPS)])

    @pl.when(cid == 1)
    def _():
        pltpu.sync_copy(acc.at[pl.ds(sid * _NPS, _NPS)], cnt1_hbm.at[pl.ds(sid * _NPS, _NPS)])


def _make_sc_scatter(depth, nbuf, concat_out):
    if concat_out:
        out_type = [jax.ShapeDtypeStruct((_NPAD, 2 * depth), jnp.float32)]
    else:
        out_type = [
            jax.ShapeDtypeStruct((_NPAD, depth), jnp.float32),
            jax.ShapeDtypeStruct((_NPAD, depth), jnp.float32),
        ]

    @functools.partial(
        pl.kernel,
        out_type=out_type,
        mesh=_mesh,
        compiler_params=pltpu.CompilerParams(use_tc_tiling_on_sc=False),
        scratch_types=(
            [
                pltpu.VMEM((_EROWS_W, _ECHUNK), jnp.int32),
                pltpu.VMEM((_EROWS_W, _ECHUNK), jnp.int32),
            ]
            + [pltpu.VMEM((_ECHUNK, depth), jnp.float32)] * nbuf
            + [pltpu.VMEM_SHARED((_NPAD, depth), jnp.float32)]
            + [pltpu.SemaphoreType.DMA] * nbuf
        ),
    )
    def _sc_scatter(edge_hbm, y_hbm, zd_hbm, *rest):
        if concat_out:
            p01_hbm = rest[0]
            rest = rest[1:]
        else:
            p0_hbm, p1_hbm = rest[:2]
            rest = rest[2:]
        rowv, colv = rest[:2]
        bufs = rest[2:2 + nbuf]
        acc = rest[2 + nbuf]
        sems = rest[3 + nbuf:]
        cid = lax.axis_index("c")
        sid = lax.axis_index("s")
        wid = sid * _NC + cid

        pltpu.sync_copy(zd_hbm.at[pl.ds(sid * _NPS, _NPS)],
                        acc.at[pl.ds(sid * _NPS, _NPS)])
        pltpu.sync_copy(edge_hbm.at[0, pl.ds(wid * _EROWS_W, _EROWS_W)], rowv)
        pltpu.sync_copy(edge_hbm.at[1, pl.ds(wid * _EROWS_W, _EROWS_W)], colv)
        plsc.subcore_barrier()

        for b in range(nbuf):
            pltpu.async_copy(y_hbm.at[rowv.at[b]], bufs[b], sems[b])

        def body(i, carry):
            for b in range(nbuf):
                j = nbuf * i + b
                pltpu.make_async_copy(y_hbm.at[rowv.at[j]], bufs[b], sems[b]).wait()
                pltpu.sync_copy(bufs[b], acc.at[colv.at[j]], add=True)

                @pl.when(j + nbuf < _EROWS_W)
                def _():
                    pltpu.async_copy(y_hbm.at[rowv.at[j + nbuf]], bufs[b], sems[b])

            return carry

        lax.fori_loop(0, _EROWS_W // nbuf, body, 0)
        plsc.subcore_barrier()

        if concat_out:
            pltpu.sync_copy(
                acc.at[pl.ds(sid * _NPS, _NPS)],
                p01_hbm.at[pl.ds(sid * _NPS, _NPS), pl.ds(cid * depth, depth)])
        else:
            @pl.when(cid == 0)
            def _():
                pltpu.sync_copy(acc.at[pl.ds(sid * _NPS, _NPS)],
                                p0_hbm.at[pl.ds(sid * _NPS, _NPS)])

            @pl.when(cid == 1)
            def _():
                pltpu.sync_copy(acc.at[pl.ds(sid * _NPS, _NPS)],
                                p1_hbm.at[pl.ds(sid * _NPS, _NPS)])

    return _sc_scatter


_sc_scatter_hid = _make_sc_scatter(_HID, 2, True)
_sc_scatter_out = _make_sc_scatter(_C, 4, False)


_R = 1000
_G = _N // _R


def _tc1a_body(x_ref, w1_ref, w2_ref, c0_ref, c1_ref, y_ref, dinv_ref, o_ref):
    xw = jnp.dot(x_ref[...], w1_ref[...], preferred_element_type=jnp.float32)
    deg = c0_ref[...] + c1_ref[...] + 1.0
    dinv = lax.rsqrt(deg)
    y_ref[...] = xw * dinv
    dinv_ref[...] = dinv

    @pl.when(pl.program_id(0) == 0)
    def _():
        w1 = w1_ref[...]
        w2 = w2_ref[...]
        g1 = lax.dot_general(w1, w1, (((1,), (1,)), ((), ())),
                             preferred_element_type=jnp.float32)
        g2 = lax.dot_general(w2, w2, (((1,), (1,)), ((), ())),
                             preferred_element_type=jnp.float32)
        i1 = (lax.broadcasted_iota(jnp.int32, (_F_IN, _F_IN), 0)
              == lax.broadcasted_iota(jnp.int32, (_F_IN, _F_IN), 1)).astype(jnp.float32)
        i2 = (lax.broadcasted_iota(jnp.int32, (_HID, _HID), 0)
              == lax.broadcasted_iota(jnp.int32, (_HID, _HID), 1)).astype(jnp.float32)
        s1 = jnp.sum((g1 - i1) ** 2)
        s2 = jnp.sum((g2 - i2) ** 2)
        o_ref[...] = jnp.reshape(jnp.sqrt(s1) + jnp.sqrt(s2), (1, 1))


_tc1a = pl.pallas_call(
    _tc1a_body,
    grid=(_G,),
    in_specs=[
        pl.BlockSpec((_R, _F_IN), lambda i: (i, 0)),
        pl.BlockSpec((_F_IN, _HID), lambda i: (0, 0)),
        pl.BlockSpec((_HID, _C), lambda i: (0, 0)),
        pl.BlockSpec((_R, 1), lambda i: (i, 0)),
        pl.BlockSpec((_R, 1), lambda i: (i, 0)),
    ],
    out_specs=[
        pl.BlockSpec((_R, _HID), lambda i: (i, 0)),
        pl.BlockSpec((_R, 1), lambda i: (i, 0)),
        pl.BlockSpec((1, 1), lambda i: (0, 0)),
    ],
    out_shape=[
        jax.ShapeDtypeStruct((_N, _HID), jnp.float32),
        jax.ShapeDtypeStruct((_N, 1), jnp.float32),
        jax.ShapeDtypeStruct((1, 1), jnp.float32),
    ],
)


def _tc2_body(p01_ref, y1_ref, dinv_ref, b1_ref, w2_ref, z_ref):
    dinv = dinv_ref[...]
    p01 = p01_ref[...]
    out1 = (p01[:, :_HID] + p01[:, _HID:] + y1_ref[...]) * dinv + b1_ref[...]
    h = jnp.maximum(out1, 0.0)
    z_ref[...] = jnp.dot(h, w2_ref[...], preferred_element_type=jnp.float32) * dinv


_tc2 = pl.pallas_call(
    _tc2_body,
    grid=(_G,),
    in_specs=[
        pl.BlockSpec((_R, 2 * _HID), lambda i: (i, 0)),
        pl.BlockSpec((_R, _HID), lambda i: (i, 0)),
        pl.BlockSpec((_R, 1), lambda i: (i, 0)),
        pl.BlockSpec((1, _HID), lambda i: (0, 0)),
        pl.BlockSpec((_HID, _C), lambda i: (0, 0)),
    ],
    out_specs=[pl.BlockSpec((_R, _C), lambda i: (i, 0))],
    out_shape=[jax.ShapeDtypeStruct((_N, _C), jnp.float32)],
)


def _tc3_body(q0_ref, q1_ref, z2_ref, dinv_ref, b2_ref, logp_ref, xout_ref):
    xo = (q0_ref[...] + q1_ref[...] + z2_ref[...]) * dinv_ref[...] + b2_ref[...]
    m = jnp.max(xo, axis=1, keepdims=True)
    t = xo - m
    lse = jnp.log(jnp.sum(jnp.exp(t), axis=1, keepdims=True))
    logp_ref[...] = t - lse
    xout_ref[...] = xo


_tc3 = pl.pallas_call(
    _tc3_body,
    grid=(_G,),
    in_specs=[
        pl.BlockSpec((_R, _C), lambda i: (i, 0)),
        pl.BlockSpec((_R, _C), lambda i: (i, 0)),
        pl.BlockSpec((_R, _C), lambda i: (i, 0)),
        pl.BlockSpec((_R, 1), lambda i: (i, 0)),
        pl.BlockSpec((1, _C), lambda i: (0, 0)),
    ],
    out_specs=[
        pl.BlockSpec((_R, _C), lambda i: (i, 0)),
        pl.BlockSpec((_R, _C), lambda i: (i, 0)),
    ],
    out_shape=[
        jax.ShapeDtypeStruct((_N, _C), jnp.float32),
        jax.ShapeDtypeStruct((_N, _C), jnp.float32),
    ],
)


def kernel(x, edge_index, W1, b1, W2, b2):
    edge_r = edge_index.reshape(2, _EROWS, _ECHUNK)
    z64 = jnp.zeros((_NPAD, _HID), jnp.float32)
    z16 = jnp.zeros((_NPAD, _C), jnp.float32)

    cnt0, cnt1 = _sc_degree(edge_r)
    y1, dinv, orto = _tc1a(x, W1, W2, cnt0.reshape(_NPAD, 1)[: _N],
                           cnt1.reshape(_NPAD, 1)[: _N])
    (p01,) = _sc_scatter_hid(edge_r, y1, z64)
    (z2,) = _tc2(p01, y1, dinv, b1.reshape(1, _HID), W2)
    q0, q1 = _sc_scatter_out(edge_r, z2, z16)
    logp, xout = _tc3(q0, q1, z2, dinv, b2.reshape(1, _C))
    return (logp, xout, orto.reshape(()))
